# Initial kernel scaffold; baseline (speedup 1.0000x reference)
import functools

import jax
import jax.numpy as jnp
from jax import lax
from jax.experimental import pallas as pl
from jax.experimental.pallas import tpu as pltpu
from jax.experimental.pallas import tpu_sc as plsc

_N = 10000
_E = 320000
_D = 128
_H = 64
_H2 = 32
_O = 2
_G = 64

_NC = 2
_NS = 16
_NW = _NC * _NS
_EPW = _E // _NW
_K = 125
_NCHUNK = _EPW // _K
_RPW = _N // _NS

_HIGH = lax.Precision.HIGHEST


def _sc_agg_body(feat, with_deg, *refs):
    if with_deg:
        (y_hbm, src_hbm, dst_hbm, zeros_hbm, ones_hbm, out_agg, out_deg,
         src_v, dst_v, rows_v, ones_v, sem,
         agg_sh, deg_sh) = refs
    else:
        (y_hbm, src_hbm, dst_hbm, zeros_hbm, out_agg,
         src_v, dst_v, rows_v, sem,
         agg_sh) = refs

    c = lax.axis_index("c")
    s = lax.axis_index("s")
    wid = c * _NS + s

    rbase = s * _RPW
    pltpu.sync_copy(zeros_hbm.at[pl.ds(rbase, _RPW), pl.ds(0, feat)],
                    agg_sh.at[pl.ds(rbase, _RPW)])
    if with_deg:
        pltpu.sync_copy(zeros_hbm.at[pl.ds(rbase, _RPW), pl.ds(0, 16)],
                        deg_sh.at[pl.ds(rbase, _RPW)])
        pltpu.sync_copy(ones_hbm, ones_v)
    pltpu.sync_copy(src_hbm.at[wid], src_v)
    pltpu.sync_copy(dst_hbm.at[wid], dst_v)
    plsc.subcore_barrier()

    def chunk(j, carry):
        pltpu.async_copy(y_hbm.at[src_v.at[j]], rows_v, sem).wait()
        pltpu.sync_copy(rows_v, agg_sh.at[dst_v.at[j]], add=True)
        if with_deg:
            pltpu.sync_copy(ones_v, deg_sh.at[dst_v.at[j]], add=True)
        return carry

    lax.fori_loop(0, _NCHUNK, chunk, 0)
    plsc.subcore_barrier()

    pltpu.sync_copy(agg_sh.at[pl.ds(rbase, _RPW)],
                    out_agg.at[c, pl.ds(rbase, _RPW)])
    if with_deg:
        pltpu.sync_copy(deg_sh.at[pl.ds(rbase, _RPW)],
                        out_deg.at[c, pl.ds(rbase, _RPW)])


def _make_sc_agg(feat, with_deg):
    mesh = plsc.VectorSubcoreMesh(core_axis_name="c", subcore_axis_name="s")
    out_type = [jax.ShapeDtypeStruct((_NC, _N, feat), jnp.float32)]
    scratch = [
        pltpu.VMEM((_NCHUNK, _K), jnp.int32),
        pltpu.VMEM((_NCHUNK, _K), jnp.int32),
        pltpu.VMEM((_K, feat), jnp.float32),
    ]
    if with_deg:
        out_type.append(jax.ShapeDtypeStruct((_NC, _N, 16), jnp.float32))
        scratch.append(pltpu.VMEM((_K, 16), jnp.float32))
    scratch.append(pltpu.SemaphoreType.DMA)
    scratch.append(pltpu.VMEM_SHARED((_N, feat), jnp.float32))
    if with_deg:
        scratch.append(pltpu.VMEM_SHARED((_N, 16), jnp.float32))

    return pl.kernel(
        functools.partial(_sc_agg_body, feat, with_deg),
        out_type=tuple(out_type),
        mesh=mesh,
        scratch_types=tuple(scratch),
    )


def _tc1_body(x_ref, wn_ref, ws_ref, y_ref, xs_ref):
    x = x_ref[...]
    y_ref[...] = jnp.dot(x, wn_ref[...], preferred_element_type=jnp.float32,
                         precision=_HIGH)
    xs_ref[...] = jnp.dot(x, ws_ref[...], preferred_element_type=jnp.float32,
                          precision=_HIGH)


def _tc2_body(xs_ref, aggp_ref, degp_ref, b1_ref, wn_ref, ws_ref,
              y2_ref, hs_ref):
    agg = aggp_ref[0] + aggp_ref[1]
    deg = degp_ref[0, :, 0:1] + degp_ref[1, :, 0:1]
    mean = agg / jnp.maximum(deg, 1.0)
    h = jnp.maximum(xs_ref[...] + mean + b1_ref[...][None, :], 0.0)
    y2_ref[...] = jnp.dot(h, wn_ref[...], preferred_element_type=jnp.float32,
                          precision=_HIGH)
    hs_ref[...] = jnp.dot(h, ws_ref[...], preferred_element_type=jnp.float32,
                          precision=_HIGH)


def _tc3_body(hs_ref, aggp_ref, degp_ref, b2_ref, batch_ref, wfc_ref,
              bfc_ref, out_ref):
    agg = aggp_ref[0] + aggp_ref[1]
    deg = degp_ref[0, :, 0:1] + degp_ref[1, :, 0:1]
    mean = agg / jnp.maximum(deg, 1.0)
    h2 = jnp.maximum(hs_ref[...] + mean + b2_ref[...][None, :], 0.0)
    gids = lax.broadcasted_iota(jnp.int32, (_G, _N), 0)
    p = (batch_ref[...] == gids).astype(jnp.float32)
    sums = jnp.dot(p, h2, preferred_element_type=jnp.float32,
                   precision=_HIGH)
    counts = jnp.sum(p, axis=1, keepdims=True)
    g = sums / jnp.maximum(counts, 1.0)
    logits = jnp.dot(g, wfc_ref[...], preferred_element_type=jnp.float32,
                     precision=_HIGH) + bfc_ref[...][None, :]
    m = jnp.max(logits, axis=1, keepdims=True)
    lse = m + jnp.log(jnp.sum(jnp.exp(logits - m), axis=1, keepdims=True))
    out_ref[...] = logits - lse


_tc1 = pl.pallas_call(
    _tc1_body,
    out_shape=(jax.ShapeDtypeStruct((_N, _H), jnp.float32),
               jax.ShapeDtypeStruct((_N, _H), jnp.float32)),
)

_tc2 = pl.pallas_call(
    _tc2_body,
    out_shape=(jax.ShapeDtypeStruct((_N, _H2), jnp.float32),
               jax.ShapeDtypeStruct((_N, _H2), jnp.float32)),
)

_tc3 = pl.pallas_call(
    _tc3_body,
    out_shape=jax.ShapeDtypeStruct((_G, _O), jnp.float32),
)

_sc_agg_deg = _make_sc_agg(_H, True)
_sc_agg = _make_sc_agg(_H2, False)


def kernel(x, edge_index, batch, W1_self, W1_neigh, b1, W2_self, W2_neigh,
           b2, W_fc, b_fc):
    src = edge_index[0].reshape(_NW, _NCHUNK, _K)
    dst = edge_index[1].reshape(_NW, _NCHUNK, _K)
    zeros = jnp.zeros((_N, _H), jnp.float32)
    ones16 = jnp.ones((_K, 16), jnp.float32)
    batch2d = batch.reshape(1, _N)

    y1, xs = _tc1(x, W1_neigh, W1_self)
    aggp1, degp = _sc_agg_deg(y1, src, dst, zeros, ones16)
    y2, hs = _tc2(xs, aggp1, degp, b1, W2_neigh, W2_self)
    aggp2, = _sc_agg(y2, src, dst, zeros)
    return _tc3(hs, aggp2, degp, b2, batch2d, W_fc, b_fc)


# speedup vs baseline: 11.1568x; 11.1568x over previous
import functools

import jax
import jax.numpy as jnp
from jax import lax
from jax.experimental import pallas as pl
from jax.experimental.pallas import tpu as pltpu
from jax.experimental.pallas import tpu_sc as plsc

_N = 10000
_E = 320000
_D = 128
_H = 64
_H2 = 32
_O = 2
_G = 64

_NC = 2
_NS = 16
_NW = _NC * _NS
_EPW = _E // _NW
_K = 125
_NCHUNK = _EPW // _K
_NP = 10240
_RPT = _NP // _NS

_HIGH = lax.Precision.HIGHEST


def _sc_agg_body(feat, with_deg, *refs):
    if with_deg:
        (y_hbm, src_hbm, dst_hbm, zagg_hbm, zdeg_hbm, ones_hbm,
         out_agg, out_deg,
         src_v, dst_v, rows_v, ones_v, sem,
         agg_sh, deg_sh) = refs
    else:
        (y_hbm, src_hbm, dst_hbm, zagg_hbm, out_agg,
         src_v, dst_v, rows_v, sem,
         agg_sh) = refs

    c = lax.axis_index("c")
    s = lax.axis_index("s")
    wid = c * _NS + s

    rbase = s * _RPT
    pltpu.sync_copy(zagg_hbm.at[pl.ds(rbase, _RPT)],
                    agg_sh.at[pl.ds(rbase, _RPT)])
    if with_deg:
        pltpu.sync_copy(zdeg_hbm.at[pl.ds(rbase, _RPT)],
                        deg_sh.at[pl.ds(rbase, _RPT)])
        pltpu.sync_copy(ones_hbm, ones_v)
    pltpu.sync_copy(src_hbm.at[wid], src_v)
    pltpu.sync_copy(dst_hbm.at[wid], dst_v)
    plsc.subcore_barrier()

    def chunk(j, carry):
        pltpu.async_copy(y_hbm.at[src_v.at[j]], rows_v, sem).wait()
        pltpu.sync_copy(rows_v, agg_sh.at[dst_v.at[j]], add=True)
        if with_deg:
            pltpu.sync_copy(ones_v, deg_sh.at[dst_v.at[j]], add=True)
        return carry

    lax.fori_loop(0, _NCHUNK, chunk, 0)
    plsc.subcore_barrier()

    pltpu.sync_copy(agg_sh.at[pl.ds(rbase, _RPT)],
                    out_agg.at[c, pl.ds(rbase, _RPT)])
    if with_deg:
        pltpu.sync_copy(deg_sh.at[pl.ds(rbase, _RPT)],
                        out_deg.at[c, pl.ds(rbase, _RPT)])


@functools.lru_cache(maxsize=None)
def _make_sc_agg(feat, with_deg):
    mesh = plsc.VectorSubcoreMesh(core_axis_name="c", subcore_axis_name="s",
                                  num_cores=_NC, num_subcores=_NS)
    out_type = [jax.ShapeDtypeStruct((_NC, _NP, feat), jnp.float32)]
    scratch = [
        pltpu.VMEM((_NCHUNK, _K), jnp.int32),
        pltpu.VMEM((_NCHUNK, _K), jnp.int32),
        pltpu.VMEM((_K, feat), jnp.float32),
    ]
    if with_deg:
        out_type.append(jax.ShapeDtypeStruct((_NC, _NP, 16), jnp.float32))
        scratch.append(pltpu.VMEM((_K, 16), jnp.float32))
    scratch.append(pltpu.SemaphoreType.DMA)
    scratch.append(pltpu.VMEM_SHARED((_NP, feat), jnp.float32))
    if with_deg:
        scratch.append(pltpu.VMEM_SHARED((_NP, 16), jnp.float32))

    return pl.kernel(
        functools.partial(_sc_agg_body, feat, with_deg),
        out_type=tuple(out_type),
        mesh=mesh,
        scratch_types=tuple(scratch),
        compiler_params=pltpu.CompilerParams(use_tc_tiling_on_sc=False),
    )


def _tc1_body(x_ref, wn_ref, ws_ref, y_ref, xs_ref):
    x = x_ref[...]
    y_ref[...] = jnp.dot(x, wn_ref[...], preferred_element_type=jnp.float32,
                         precision=_HIGH)
    xs_ref[...] = jnp.dot(x, ws_ref[...], preferred_element_type=jnp.float32,
                          precision=_HIGH)


def _tc2_body(xs_ref, aggp_ref, degp_ref, b1_ref, wn_ref, ws_ref,
              y2_ref, hs_ref):
    agg = aggp_ref[0, :_N] + aggp_ref[1, :_N]
    deg = degp_ref[0, :_N, 0:1] + degp_ref[1, :_N, 0:1]
    mean = agg / jnp.maximum(deg, 1.0)
    h = jnp.maximum(xs_ref[...] + mean + b1_ref[...][None, :], 0.0)
    y2_ref[...] = jnp.dot(h, wn_ref[...], preferred_element_type=jnp.float32,
                          precision=_HIGH)
    hs_ref[...] = jnp.dot(h, ws_ref[...], preferred_element_type=jnp.float32,
                          precision=_HIGH)


def _tc3_body(hs_ref, aggp_ref, degp_ref, b2_ref, batch_ref, wfc_ref,
              bfc_ref, out_ref):
    agg = aggp_ref[0, :_N] + aggp_ref[1, :_N]
    deg = degp_ref[0, :_N, 0:1] + degp_ref[1, :_N, 0:1]
    mean = agg / jnp.maximum(deg, 1.0)
    h2 = jnp.maximum(hs_ref[...] + mean + b2_ref[...][None, :], 0.0)
    gids = lax.broadcasted_iota(jnp.int32, (_G, _N), 0)
    p = (batch_ref[...] == gids).astype(jnp.float32)
    sums = jnp.dot(p, h2, preferred_element_type=jnp.float32,
                   precision=_HIGH)
    counts = jnp.sum(p, axis=1, keepdims=True)
    g = sums / jnp.maximum(counts, 1.0)
    logits = jnp.dot(g, wfc_ref[...], preferred_element_type=jnp.float32,
                     precision=_HIGH) + bfc_ref[...][None, :]
    m = jnp.max(logits, axis=1, keepdims=True)
    lse = m + jnp.log(jnp.sum(jnp.exp(logits - m), axis=1, keepdims=True))
    out_ref[...] = logits - lse


_tc1 = pl.pallas_call(
    _tc1_body,
    out_shape=(jax.ShapeDtypeStruct((_N, _H), jnp.float32),
               jax.ShapeDtypeStruct((_N, _H), jnp.float32)),
)

_tc2 = pl.pallas_call(
    _tc2_body,
    out_shape=(jax.ShapeDtypeStruct((_N, _H2), jnp.float32),
               jax.ShapeDtypeStruct((_N, _H2), jnp.float32)),
)

_tc3 = pl.pallas_call(
    _tc3_body,
    out_shape=jax.ShapeDtypeStruct((_G, _O), jnp.float32),
)

def kernel(x, edge_index, batch, W1_self, W1_neigh, b1, W2_self, W2_neigh,
           b2, W_fc, b_fc):
    src = edge_index[0].reshape(_NW, _NCHUNK, _K)
    dst = edge_index[1].reshape(_NW, _NCHUNK, _K)
    zeros_h = jnp.zeros((_NP, _H), jnp.float32)
    zeros_h2 = jnp.zeros((_NP, _H2), jnp.float32)
    zeros_16 = jnp.zeros((_NP, 16), jnp.float32)
    ones16 = jnp.ones((_K, 16), jnp.float32)
    batch2d = batch.reshape(1, _N)

    y1, xs = _tc1(x, W1_neigh, W1_self)
    aggp1, degp = _make_sc_agg(_H, True)(y1, src, dst, zeros_h, zeros_16,
                                         ones16)
    y2, hs = _tc2(xs, aggp1, degp, b1, W2_neigh, W2_self)
    aggp2, = _make_sc_agg(_H2, False)(y2, src, dst, zeros_h2)
    return _tc3(hs, aggp2, degp, b2, batch2d, W_fc, b_fc)


# speedup vs baseline: 17.3793x; 1.5577x over previous
import functools

import jax
import jax.numpy as jnp
from jax import lax
from jax.experimental import pallas as pl
from jax.experimental.pallas import tpu as pltpu
from jax.experimental.pallas import tpu_sc as plsc

_N = 10000
_E = 320000
_D = 128
_H = 64
_H2 = 32
_O = 2
_G = 64

_NC = 2
_NS = 16
_NW = _NC * _NS
_EPW = _E // _NW
_K = 125
_NCHUNK = _EPW // _K
_NBUF = 4
_NP = 10240
_RPT = _NP // _NS

_HIGH = lax.Precision.HIGHEST


def _sc_agg_body(feat, with_deg, *refs):
    if with_deg:
        (y_hbm, src_hbm, dst_hbm, zagg_hbm, zdeg_hbm, ones_hbm,
         out_agg, out_deg,
         src_v, dst_v, rows_v, ones_v, sem,
         agg_sh, deg_sh) = refs
    else:
        (y_hbm, src_hbm, dst_hbm, zagg_hbm, out_agg,
         src_v, dst_v, rows_v, sem,
         agg_sh) = refs

    c = lax.axis_index("c")
    s = lax.axis_index("s")
    wid = c * _NS + s

    rbase = s * _RPT
    pltpu.sync_copy(zagg_hbm.at[pl.ds(rbase, _RPT)],
                    agg_sh.at[pl.ds(rbase, _RPT)])
    if with_deg:
        pltpu.sync_copy(zdeg_hbm.at[pl.ds(rbase, _RPT)],
                        deg_sh.at[pl.ds(rbase, _RPT)])
        pltpu.sync_copy(ones_hbm, ones_v)
    pltpu.sync_copy(src_hbm.at[wid], src_v)
    pltpu.sync_copy(dst_hbm.at[wid], dst_v)
    plsc.subcore_barrier()

    def fire(j, b):
        pltpu.async_copy(y_hbm.at[src_v.at[j]], rows_v.at[b], sem.at[b])

    for b in range(_NBUF - 1):
        fire(b, b)

    def chunk(j, carry):
        b = lax.rem(j, _NBUF)
        pltpu.make_async_copy(y_hbm.at[src_v.at[j]], rows_v.at[b],
                              sem.at[b]).wait()
        jn = j + _NBUF - 1

        @pl.when(jn < _NCHUNK)
        def _():
            fire(jn, lax.rem(jn, _NBUF))

        pltpu.sync_copy(rows_v.at[b], agg_sh.at[dst_v.at[j]], add=True)
        if with_deg:
            pltpu.sync_copy(ones_v, deg_sh.at[dst_v.at[j]], add=True)
        return carry

    lax.fori_loop(0, _NCHUNK, chunk, 0)
    plsc.subcore_barrier()

    pltpu.sync_copy(agg_sh.at[pl.ds(rbase, _RPT)],
                    out_agg.at[c, pl.ds(rbase, _RPT)])
    if with_deg:
        pltpu.sync_copy(deg_sh.at[pl.ds(rbase, _RPT)],
                        out_deg.at[c, pl.ds(rbase, _RPT)])


@functools.lru_cache(maxsize=None)
def _make_sc_agg(feat, with_deg):
    mesh = plsc.VectorSubcoreMesh(core_axis_name="c", subcore_axis_name="s",
                                  num_cores=_NC, num_subcores=_NS)
    out_type = [jax.ShapeDtypeStruct((_NC, _NP, feat), jnp.float32)]
    scratch = [
        pltpu.VMEM((_NCHUNK, _K), jnp.int32),
        pltpu.VMEM((_NCHUNK, _K), jnp.int32),
        pltpu.VMEM((_NBUF, _K, feat), jnp.float32),
    ]
    if with_deg:
        out_type.append(jax.ShapeDtypeStruct((_NC, _NP, 16), jnp.float32))
        scratch.append(pltpu.VMEM((_K, 16), jnp.float32))
    scratch.append(pltpu.SemaphoreType.DMA((_NBUF,)))
    scratch.append(pltpu.VMEM_SHARED((_NP, feat), jnp.float32))
    if with_deg:
        scratch.append(pltpu.VMEM_SHARED((_NP, 16), jnp.float32))

    return pl.kernel(
        functools.partial(_sc_agg_body, feat, with_deg),
        out_type=tuple(out_type),
        mesh=mesh,
        scratch_types=tuple(scratch),
        compiler_params=pltpu.CompilerParams(use_tc_tiling_on_sc=False),
    )


def _tc1_body(x_ref, wn_ref, ws_ref, y_ref, xs_ref):
    x = x_ref[...]
    y_ref[...] = jnp.dot(x, wn_ref[...], preferred_element_type=jnp.float32,
                         precision=_HIGH)
    xs_ref[...] = jnp.dot(x, ws_ref[...], preferred_element_type=jnp.float32,
                          precision=_HIGH)


def _tc2_body(xs_ref, aggp_ref, degp_ref, b1_ref, wn_ref, ws_ref,
              y2_ref, hs_ref):
    agg = aggp_ref[0, :_N] + aggp_ref[1, :_N]
    deg = degp_ref[0, :_N, 0:1] + degp_ref[1, :_N, 0:1]
    mean = agg / jnp.maximum(deg, 1.0)
    h = jnp.maximum(xs_ref[...] + mean + b1_ref[...][None, :], 0.0)
    y2_ref[...] = jnp.dot(h, wn_ref[...], preferred_element_type=jnp.float32,
                          precision=_HIGH)
    hs_ref[...] = jnp.dot(h, ws_ref[...], preferred_element_type=jnp.float32,
                          precision=_HIGH)


def _tc3_body(hs_ref, aggp_ref, degp_ref, b2_ref, batch_ref, wfc_ref,
              bfc_ref, out_ref):
    agg = aggp_ref[0, :_N] + aggp_ref[1, :_N]
    deg = degp_ref[0, :_N, 0:1] + degp_ref[1, :_N, 0:1]
    mean = agg / jnp.maximum(deg, 1.0)
    h2 = jnp.maximum(hs_ref[...] + mean + b2_ref[...][None, :], 0.0)
    gids = lax.broadcasted_iota(jnp.int32, (_G, _N), 0)
    p = (batch_ref[...] == gids).astype(jnp.float32)
    sums = jnp.dot(p, h2, preferred_element_type=jnp.float32,
                   precision=_HIGH)
    counts = jnp.sum(p, axis=1, keepdims=True)
    g = sums / jnp.maximum(counts, 1.0)
    logits = jnp.dot(g, wfc_ref[...], preferred_element_type=jnp.float32,
                     precision=_HIGH) + bfc_ref[...][None, :]
    m = jnp.max(logits, axis=1, keepdims=True)
    lse = m + jnp.log(jnp.sum(jnp.exp(logits - m), axis=1, keepdims=True))
    out_ref[...] = logits - lse


_tc1 = pl.pallas_call(
    _tc1_body,
    out_shape=(jax.ShapeDtypeStruct((_N, _H), jnp.float32),
               jax.ShapeDtypeStruct((_N, _H), jnp.float32)),
)

_tc2 = pl.pallas_call(
    _tc2_body,
    out_shape=(jax.ShapeDtypeStruct((_N, _H2), jnp.float32),
               jax.ShapeDtypeStruct((_N, _H2), jnp.float32)),
)

_tc3 = pl.pallas_call(
    _tc3_body,
    out_shape=jax.ShapeDtypeStruct((_G, _O), jnp.float32),
)

def kernel(x, edge_index, batch, W1_self, W1_neigh, b1, W2_self, W2_neigh,
           b2, W_fc, b_fc):
    src = edge_index[0].reshape(_NW, _NCHUNK, _K)
    dst = edge_index[1].reshape(_NW, _NCHUNK, _K)
    zeros_h = jnp.zeros((_NP, _H), jnp.float32)
    zeros_h2 = jnp.zeros((_NP, _H2), jnp.float32)
    zeros_16 = jnp.zeros((_NP, 16), jnp.float32)
    ones16 = jnp.ones((_K, 16), jnp.float32)
    batch2d = batch.reshape(1, _N)

    y1, xs = _tc1(x, W1_neigh, W1_self)
    aggp1, degp = _make_sc_agg(_H, True)(y1, src, dst, zeros_h, zeros_16,
                                         ones16)
    y2, hs = _tc2(xs, aggp1, degp, b1, W2_neigh, W2_self)
    aggp2, = _make_sc_agg(_H2, False)(y2, src, dst, zeros_h2)
    return _tc3(hs, aggp2, degp, b2, batch2d, W_fc, b_fc)


# speedup vs baseline: 24.1076x; 1.3871x over previous
import functools

import jax
import jax.numpy as jnp
from jax import lax
from jax.experimental import pallas as pl
from jax.experimental.pallas import tpu as pltpu
from jax.experimental.pallas import tpu_sc as plsc

_N = 10000
_E = 320000
_D = 128
_H = 64
_H2 = 32
_O = 2
_G = 64

_NC = 2
_NS = 16
_NW = _NC * _NS
_EPW = _E // _NW
_K = 1000
_NCHUNK = _EPW // _K
_NBUF = 2
_NP = 10240
_RPT = _NP // _NS
_DW = 32


def _sc_agg_body(feat, with_deg, *refs):
    if with_deg:
        (y_hbm, ei_hbm, zagg_hbm, zdeg_hbm, ones_hbm,
         out_agg, out_deg,
         src_v, dst_v, rows_v, ones_v, sem, isem,
         agg_sh, deg_sh) = refs
    else:
        (y_hbm, ei_hbm, zagg_hbm, out_agg,
         src_v, dst_v, rows_v, sem, isem,
         agg_sh) = refs

    c = lax.axis_index("c")
    s = lax.axis_index("s")
    wid = c * _NS + s
    ebase = wid * _EPW

    for j in range(_NCHUNK):
        pltpu.async_copy(ei_hbm.at[0, pl.ds(ebase + j * _K, _K)],
                         src_v.at[j], isem)
        pltpu.async_copy(ei_hbm.at[1, pl.ds(ebase + j * _K, _K)],
                         dst_v.at[j], isem)

    rbase = s * _RPT
    pltpu.sync_copy(zagg_hbm.at[pl.ds(rbase, _RPT)],
                    agg_sh.at[pl.ds(rbase, _RPT)])
    if with_deg:
        pltpu.sync_copy(zdeg_hbm.at[pl.ds(rbase, _RPT)],
                        deg_sh.at[pl.ds(rbase, _RPT)])
        pltpu.sync_copy(ones_hbm, ones_v)
    for j in range(_NCHUNK):
        pltpu.make_async_copy(ei_hbm.at[0, pl.ds(ebase + j * _K, _K)],
                              src_v.at[j], isem).wait()
        pltpu.make_async_copy(ei_hbm.at[1, pl.ds(ebase + j * _K, _K)],
                              dst_v.at[j], isem).wait()
    plsc.subcore_barrier()

    def fire(j, b):
        pltpu.async_copy(y_hbm.at[src_v.at[j]], rows_v.at[b], sem.at[b])

    for b in range(_NBUF - 1):
        fire(b, b)

    def chunk(j, carry):
        b = lax.rem(j, _NBUF)
        pltpu.make_async_copy(y_hbm.at[src_v.at[j]], rows_v.at[b],
                              sem.at[b]).wait()
        jn = j + _NBUF - 1

        @pl.when(jn < _NCHUNK)
        def _():
            fire(jn, lax.rem(jn, _NBUF))

        pltpu.sync_copy(rows_v.at[b], agg_sh.at[dst_v.at[j]], add=True)
        if with_deg:
            pltpu.sync_copy(ones_v, deg_sh.at[dst_v.at[j]], add=True)
        return carry

    lax.fori_loop(0, _NCHUNK, chunk, 0)
    plsc.subcore_barrier()

    pltpu.sync_copy(agg_sh.at[pl.ds(rbase, _RPT)],
                    out_agg.at[c, pl.ds(rbase, _RPT)])
    if with_deg:
        pltpu.sync_copy(deg_sh.at[pl.ds(rbase, _RPT)],
                        out_deg.at[c, pl.ds(rbase, _RPT)])


@functools.lru_cache(maxsize=None)
def _make_sc_agg(feat, with_deg):
    mesh = plsc.VectorSubcoreMesh(core_axis_name="c", subcore_axis_name="s",
                                  num_cores=_NC, num_subcores=_NS)
    out_type = [jax.ShapeDtypeStruct((_NC, _NP, feat), jnp.bfloat16)]
    scratch = [
        pltpu.VMEM((_NCHUNK, _K), jnp.int32),
        pltpu.VMEM((_NCHUNK, _K), jnp.int32),
        pltpu.VMEM((_NBUF, _K, feat), jnp.bfloat16),
    ]
    if with_deg:
        out_type.append(jax.ShapeDtypeStruct((_NC, _NP, _DW), jnp.bfloat16))
        scratch.append(pltpu.VMEM((_K, _DW), jnp.bfloat16))
    scratch.append(pltpu.SemaphoreType.DMA((_NBUF,)))
    scratch.append(pltpu.SemaphoreType.DMA)
    scratch.append(pltpu.VMEM_SHARED((_NP, feat), jnp.bfloat16))
    if with_deg:
        scratch.append(pltpu.VMEM_SHARED((_NP, _DW), jnp.bfloat16))

    return pl.kernel(
        functools.partial(_sc_agg_body, feat, with_deg),
        out_type=tuple(out_type),
        mesh=mesh,
        scratch_types=tuple(scratch),
        compiler_params=pltpu.CompilerParams(use_tc_tiling_on_sc=False),
    )


def _tc1_body(x_ref, wn_ref, ws_ref, y_ref, xs_ref):
    x = x_ref[...]
    y_ref[...] = jnp.dot(x, wn_ref[...],
                         preferred_element_type=jnp.float32).astype(jnp.bfloat16)
    xs_ref[...] = jnp.dot(x, ws_ref[...], preferred_element_type=jnp.float32)


def _tc2_body(xs_ref, aggp_ref, degp_ref, b1_ref, wn_ref, ws_ref,
              y2_ref, hs_ref):
    agg = (aggp_ref[0, :_N].astype(jnp.float32)
           + aggp_ref[1, :_N].astype(jnp.float32))
    deg = (degp_ref[0, :_N, 0:1].astype(jnp.float32)
           + degp_ref[1, :_N, 0:1].astype(jnp.float32))
    mean = agg / jnp.maximum(deg, 1.0)
    h = jnp.maximum(xs_ref[...] + mean + b1_ref[...][None, :], 0.0)
    y2_ref[...] = jnp.dot(h, wn_ref[...],
                          preferred_element_type=jnp.float32).astype(jnp.bfloat16)
    hs_ref[...] = jnp.dot(h, ws_ref[...], preferred_element_type=jnp.float32)


def _tc3_body(hs_ref, aggp_ref, degp_ref, b2_ref, batch_ref, wfc_ref,
              bfc_ref, out_ref):
    agg = (aggp_ref[0, :_N].astype(jnp.float32)
           + aggp_ref[1, :_N].astype(jnp.float32))
    deg = (degp_ref[0, :_N, 0:1].astype(jnp.float32)
           + degp_ref[1, :_N, 0:1].astype(jnp.float32))
    mean = agg / jnp.maximum(deg, 1.0)
    h2 = jnp.maximum(hs_ref[...] + mean + b2_ref[...][None, :], 0.0)
    gids = lax.broadcasted_iota(jnp.int32, (_G, _N), 0)
    p = (batch_ref[...] == gids).astype(jnp.float32)
    sums = jnp.dot(p, h2, preferred_element_type=jnp.float32)
    counts = jnp.sum(p, axis=1, keepdims=True)
    g = sums / jnp.maximum(counts, 1.0)
    logits = (jnp.dot(g, wfc_ref[...], preferred_element_type=jnp.float32)
              + bfc_ref[...][None, :])
    m = jnp.max(logits, axis=1, keepdims=True)
    lse = m + jnp.log(jnp.sum(jnp.exp(logits - m), axis=1, keepdims=True))
    out_ref[...] = logits - lse


_tc1 = pl.pallas_call(
    _tc1_body,
    out_shape=(jax.ShapeDtypeStruct((_N, _H), jnp.bfloat16),
               jax.ShapeDtypeStruct((_N, _H), jnp.float32)),
)

_tc2 = pl.pallas_call(
    _tc2_body,
    out_shape=(jax.ShapeDtypeStruct((_N, _H2), jnp.bfloat16),
               jax.ShapeDtypeStruct((_N, _H2), jnp.float32)),
)

_tc3 = pl.pallas_call(
    _tc3_body,
    out_shape=jax.ShapeDtypeStruct((_G, _O), jnp.float32),
)


def kernel(x, edge_index, batch, W1_self, W1_neigh, b1, W2_self, W2_neigh,
           b2, W_fc, b_fc):
    zeros_h = jnp.zeros((_NP, _H), jnp.bfloat16)
    zeros_h2 = jnp.zeros((_NP, _H2), jnp.bfloat16)
    zeros_d = jnp.zeros((_NP, _DW), jnp.bfloat16)
    ones_d = jnp.ones((_K, _DW), jnp.bfloat16)
    batch2d = batch.reshape(1, _N)

    y1, xs = _tc1(x, W1_neigh, W1_self)
    aggp1, degp = _make_sc_agg(_H, True)(y1, edge_index, zeros_h, zeros_d,
                                         ones_d)
    y2, hs = _tc2(xs, aggp1, degp, b1, W2_neigh, W2_self)
    aggp2, = _make_sc_agg(_H2, False)(y2, edge_index, zeros_h2)
    return _tc3(hs, aggp2, degp, b2, batch2d, W_fc, b_fc)


# speedup vs baseline: 25.4261x; 1.0547x over previous
import functools

import jax
import jax.numpy as jnp
from jax import lax
from jax.experimental import pallas as pl
from jax.experimental.pallas import tpu as pltpu
from jax.experimental.pallas import tpu_sc as plsc

_N = 10000
_E = 320000
_D = 128
_H = 64
_H2 = 32
_O = 2
_G = 64

_NC = 2
_NS = 16
_NW = _NC * _NS
_EPW = _E // _NW
_K = 1000
_NCHUNK = _EPW // _K

_NP = 10240
_RPT = _NP // _NS
_DW = 16


def _sc_agg_body(feat, with_deg, nbuf, *refs):
    if with_deg:
        (y_hbm, ei_hbm, zagg_hbm, zdeg_hbm, ones_hbm,
         out_agg, out_deg,
         src_v, dst_v, rows_v, ones_v, sem, isem,
         agg_sh, deg_sh) = refs
    else:
        (y_hbm, ei_hbm, zagg_hbm, out_agg,
         src_v, dst_v, rows_v, sem, isem,
         agg_sh) = refs

    c = lax.axis_index("c")
    s = lax.axis_index("s")
    wid = c * _NS + s
    ebase = wid * _EPW

    for j in range(_NCHUNK):
        pltpu.async_copy(ei_hbm.at[0, pl.ds(ebase + j * _K, _K)],
                         src_v.at[j], isem)
        pltpu.async_copy(ei_hbm.at[1, pl.ds(ebase + j * _K, _K)],
                         dst_v.at[j], isem)

    rbase = s * _RPT
    pltpu.sync_copy(zagg_hbm.at[pl.ds(rbase, _RPT)],
                    agg_sh.at[pl.ds(rbase, _RPT)])
    if with_deg:
        pltpu.sync_copy(zdeg_hbm.at[pl.ds(rbase, _RPT)],
                        deg_sh.at[pl.ds(rbase, _RPT)])
        pltpu.sync_copy(ones_hbm, ones_v)
    for j in range(_NCHUNK):
        pltpu.make_async_copy(ei_hbm.at[0, pl.ds(ebase + j * _K, _K)],
                              src_v.at[j], isem).wait()
        pltpu.make_async_copy(ei_hbm.at[1, pl.ds(ebase + j * _K, _K)],
                              dst_v.at[j], isem).wait()
    plsc.subcore_barrier()

    def fire(j, b):
        pltpu.async_copy(y_hbm.at[src_v.at[j]], rows_v.at[b], sem.at[b])

    for b in range(nbuf - 1):
        fire(b, b)

    def chunk(j, carry):
        b = lax.rem(j, nbuf)
        pltpu.make_async_copy(y_hbm.at[src_v.at[j]], rows_v.at[b],
                              sem.at[b]).wait()
        jn = j + nbuf - 1

        @pl.when(jn < _NCHUNK)
        def _():
            fire(jn, lax.rem(jn, nbuf))

        pltpu.sync_copy(rows_v.at[b], agg_sh.at[dst_v.at[j]], add=True)
        if with_deg:
            pltpu.sync_copy(ones_v, deg_sh.at[dst_v.at[j]], add=True)
        return carry

    lax.fori_loop(0, _NCHUNK, chunk, 0)
    plsc.subcore_barrier()

    pltpu.sync_copy(agg_sh.at[pl.ds(rbase, _RPT)],
                    out_agg.at[c, pl.ds(rbase, _RPT)])
    if with_deg:
        pltpu.sync_copy(deg_sh.at[pl.ds(rbase, _RPT)],
                        out_deg.at[c, pl.ds(rbase, _RPT)])


@functools.lru_cache(maxsize=None)
def _make_sc_agg(feat, with_deg, nbuf):
    mesh = plsc.VectorSubcoreMesh(core_axis_name="c", subcore_axis_name="s",
                                  num_cores=_NC, num_subcores=_NS)
    out_type = [jax.ShapeDtypeStruct((_NC, _NP, feat), jnp.bfloat16)]
    scratch = [
        pltpu.VMEM((_NCHUNK, _K), jnp.int32),
        pltpu.VMEM((_NCHUNK, _K), jnp.int32),
        pltpu.VMEM((nbuf, _K, feat), jnp.bfloat16),
    ]
    if with_deg:
        out_type.append(jax.ShapeDtypeStruct((_NC, _NP, _DW), jnp.bfloat16))
        scratch.append(pltpu.VMEM((_K, _DW), jnp.bfloat16))
    scratch.append(pltpu.SemaphoreType.DMA((nbuf,)))
    scratch.append(pltpu.SemaphoreType.DMA)
    scratch.append(pltpu.VMEM_SHARED((_NP, feat), jnp.bfloat16))
    if with_deg:
        scratch.append(pltpu.VMEM_SHARED((_NP, _DW), jnp.bfloat16))

    return pl.kernel(
        functools.partial(_sc_agg_body, feat, with_deg, nbuf),
        out_type=tuple(out_type),
        mesh=mesh,
        scratch_types=tuple(scratch),
        compiler_params=pltpu.CompilerParams(use_tc_tiling_on_sc=False),
    )


def _tc1_body(x_ref, wn_ref, ws_ref, y_ref, xs_ref):
    x = x_ref[...]
    y_ref[...] = jnp.dot(x, wn_ref[...],
                         preferred_element_type=jnp.float32).astype(jnp.bfloat16)
    xs_ref[...] = jnp.dot(x, ws_ref[...], preferred_element_type=jnp.float32)


def _tc2_body(xs_ref, aggp_ref, degp_ref, b1_ref, wn_ref, ws_ref,
              y2_ref, hs_ref):
    agg = (aggp_ref[0, :_N].astype(jnp.float32)
           + aggp_ref[1, :_N].astype(jnp.float32))
    deg = (degp_ref[0, :_N, 0:1].astype(jnp.float32)
           + degp_ref[1, :_N, 0:1].astype(jnp.float32))
    mean = agg / jnp.maximum(deg, 1.0)
    h = jnp.maximum(xs_ref[...] + mean + b1_ref[...][None, :], 0.0)
    y2_ref[...] = jnp.dot(h, wn_ref[...],
                          preferred_element_type=jnp.float32).astype(jnp.bfloat16)
    hs_ref[...] = jnp.dot(h, ws_ref[...], preferred_element_type=jnp.float32)


def _tc3_body(hs_ref, aggp_ref, degp_ref, b2_ref, batch_ref, wfc_ref,
              bfc_ref, out_ref):
    agg = (aggp_ref[0, :_N].astype(jnp.float32)
           + aggp_ref[1, :_N].astype(jnp.float32))
    deg = (degp_ref[0, :_N, 0:1].astype(jnp.float32)
           + degp_ref[1, :_N, 0:1].astype(jnp.float32))
    mean = agg / jnp.maximum(deg, 1.0)
    h2 = jnp.maximum(hs_ref[...] + mean + b2_ref[...][None, :], 0.0)
    gids = lax.broadcasted_iota(jnp.int32, (_G, _N), 0)
    p = (batch_ref[...] == gids).astype(jnp.float32)
    sums = jnp.dot(p, h2, preferred_element_type=jnp.float32)
    counts = jnp.sum(p, axis=1, keepdims=True)
    g = sums / jnp.maximum(counts, 1.0)
    logits = (jnp.dot(g, wfc_ref[...], preferred_element_type=jnp.float32)
              + bfc_ref[...][None, :])
    m = jnp.max(logits, axis=1, keepdims=True)
    lse = m + jnp.log(jnp.sum(jnp.exp(logits - m), axis=1, keepdims=True))
    out_ref[...] = logits - lse


_tc1 = pl.pallas_call(
    _tc1_body,
    out_shape=(jax.ShapeDtypeStruct((_N, _H), jnp.bfloat16),
               jax.ShapeDtypeStruct((_N, _H), jnp.float32)),
)

_tc2 = pl.pallas_call(
    _tc2_body,
    out_shape=(jax.ShapeDtypeStruct((_N, _H2), jnp.bfloat16),
               jax.ShapeDtypeStruct((_N, _H2), jnp.float32)),
)

_tc3 = pl.pallas_call(
    _tc3_body,
    out_shape=jax.ShapeDtypeStruct((_G, _O), jnp.float32),
)


def kernel(x, edge_index, batch, W1_self, W1_neigh, b1, W2_self, W2_neigh,
           b2, W_fc, b_fc):
    zeros_h = jnp.zeros((_NP, _H), jnp.bfloat16)
    zeros_h2 = jnp.zeros((_NP, _H2), jnp.bfloat16)
    zeros_d = jnp.zeros((_NP, _DW), jnp.bfloat16)
    ones_d = jnp.ones((_K, _DW), jnp.bfloat16)
    batch2d = batch.reshape(1, _N)

    y1, xs = _tc1(x, W1_neigh, W1_self)
    aggp1, degp = _make_sc_agg(_H, True, 2)(y1, edge_index, zeros_h,
                                            zeros_d, ones_d)
    y2, hs = _tc2(xs, aggp1, degp, b1, W2_neigh, W2_self)
    aggp2, = _make_sc_agg(_H2, False, 4)(y2, edge_index, zeros_h2)
    return _tc3(hs, aggp2, degp, b2, batch2d, W_fc, b_fc)
